# baseline (speedup 1.0000x reference)
import functools

import jax
import jax.numpy as jnp
from jax import lax
from jax.experimental import pallas as pl
from jax.experimental.pallas import tpu as pltpu
from jax.experimental.pallas import tpu_sc as plsc

_N = 10000
_NP = 10240
_E = 320000
_B = 64
_F = 32
_PK = 4
_ROWS = _NP // _PK
_CHUNK = 128
_NCHUNKS = _E // _CHUNK
_NC = 2
_NS = 16
_NW = _NC * _NS
_CPT = _NCHUNKS // _NW
_XTRA = _NCHUNKS - _CPT * _NW
_K = 6
_NGROUPS = _CPT // _K
_RPT = _NP // _NS


def _zero_rows(ref, nrows, width):
    z = jnp.zeros((16,), jnp.float32)

    def body(i, _):
        for j in range(width // 16):
            ref[i, pl.ds(j * 16, 16)] = z
        return 0

    lax.fori_loop(0, nrows, body, 0)


def _acc_zero(stage, acc, s):
    row0 = pl.multiple_of(s * _RPT, 8)
    pltpu.sync_copy(stage, acc.at[pl.ds(row0, _RPT)])


def _acc_writeback(stage, acc, out_hbm, c, s):
    row0 = pl.multiple_of(s * _RPT, 8)
    pltpu.sync_copy(acc.at[pl.ds(row0, _RPT)], stage)
    pltpu.sync_copy(stage, out_hbm.at[c, pl.ds(row0, _RPT)])


def _load_tile_indices(edge3_hbm, dim, idx2d, w):
    lo = w * _CPT
    pltpu.sync_copy(edge3_hbm.at[dim, pl.ds(lo, _CPT)],
                    idx2d.at[pl.ds(0, _CPT)])

    @pl.when(w < _XTRA)
    def _():
        pltpu.sync_copy(edge3_hbm.at[dim, pl.ds(_CPT * _NW + w, 1)],
                        idx2d.at[pl.ds(_CPT, 1)])


def _sc_degree_body(edge3_hbm, out_hbm, acc, dst2d, onesv, buf16, stage, ssem):
    c = lax.axis_index("c")
    s = lax.axis_index("s")
    w = c * _NS + s

    one = jnp.ones((16,), jnp.float32)

    def fill_ones(i, _):
        onesv[i] = one
        return 0

    lax.fori_loop(0, _CHUNK, fill_ones, 0)
    _zero_rows(buf16, _RPT, 16)
    row0 = pl.multiple_of(s * _RPT, 8)
    pltpu.sync_copy(buf16, acc.at[pl.ds(row0, _RPT)])
    _load_tile_indices(edge3_hbm, 1, dst2d, w)
    plsc.subcore_barrier()

    def group(g, _):
        descs = [pltpu.async_copy(onesv, acc.at[dst2d.at[g * _K + b]], ssem,
                                  add=True)
                 for b in range(_K)]
        for d in descs:
            d.wait()
        return 0

    lax.fori_loop(0, _NGROUPS, group, 0)

    @pl.when(w < _XTRA)
    def _():
        pltpu.sync_copy(onesv, acc.at[dst2d.at[_CPT]], add=True)

    plsc.subcore_barrier()
    pltpu.sync_copy(acc.at[pl.ds(row0, _RPT)], buf16)

    def expand(i, _):
        v = buf16[i]
        stage[i, pl.ds(0, 16)] = v
        stage[i, pl.ds(16, 16)] = v
        return 0

    lax.fori_loop(0, _RPT, expand, 0)
    pltpu.sync_copy(stage, out_hbm.at[c, pl.ds(row0, _RPT)])


def _sc_agg_body(u_hbm, edge3_hbm, out_hbm, acc, src2d, dst2d,
                 *rest):
    rows = rest[:_K]
    stage, gsem, ssem = rest[_K], rest[_K + 1], rest[_K + 2]
    c = lax.axis_index("c")
    s = lax.axis_index("s")
    w = c * _NS + s

    _zero_rows(stage, _RPT, _F)
    _acc_zero(stage, acc, s)
    _load_tile_indices(edge3_hbm, 0, src2d, w)
    _load_tile_indices(edge3_hbm, 1, dst2d, w)
    plsc.subcore_barrier()

    def group(g, _):
        gd = [pltpu.async_copy(u_hbm.at[src2d.at[g * _K + b]], rows[b], gsem)
              for b in range(_K)]
        sd = []
        for b in range(_K):
            gd[b].wait()
            sd.append(pltpu.async_copy(rows[b], acc.at[dst2d.at[g * _K + b]],
                                       ssem, add=True))
        for d in sd:
            d.wait()
        return 0

    lax.fori_loop(0, _NGROUPS, group, 0)

    @pl.when(w < _XTRA)
    def _():
        pltpu.async_copy(u_hbm.at[src2d.at[_CPT]], rows[0], gsem).wait()
        pltpu.sync_copy(rows[0], acc.at[dst2d.at[_CPT]], add=True)

    plsc.subcore_barrier()
    _acc_writeback(stage, acc, out_hbm, c, s)


@functools.lru_cache(maxsize=None)
def _sc_kernels():
    mesh = plsc.VectorSubcoreMesh(core_axis_name="c", subcore_axis_name="s",
                                  num_cores=_NC, num_subcores=_NS)
    sc_degree = pl.kernel(
        _sc_degree_body,
        out_type=jax.ShapeDtypeStruct((_NC, _NP, _F), jnp.float32),
        mesh=mesh,
        compiler_params=pltpu.CompilerParams(use_tc_tiling_on_sc=False),
        scratch_types=[
            pltpu.VMEM_SHARED((_NP, 16), jnp.float32),
            pltpu.VMEM((_CPT + 1, _CHUNK), jnp.int32),
            pltpu.VMEM((_CHUNK, 16), jnp.float32),
            pltpu.VMEM((_RPT, 16), jnp.float32),
            pltpu.VMEM((_RPT, _F), jnp.float32),
            pltpu.SemaphoreType.DMA,
        ],
    )
    sc_agg = pl.kernel(
        _sc_agg_body,
        out_type=jax.ShapeDtypeStruct((_NC, _NP, _F), jnp.float32),
        mesh=mesh,
        compiler_params=pltpu.CompilerParams(use_tc_tiling_on_sc=False),
        scratch_types=(
            [pltpu.VMEM_SHARED((_NP, _F), jnp.float32)]
            + [pltpu.VMEM((_CPT + 1, _CHUNK), jnp.int32)] * 2
            + [pltpu.VMEM((_CHUNK, _F), jnp.float32)] * _K
            + [pltpu.VMEM((_RPT, _F), jnp.float32)]
            + [pltpu.SemaphoreType.DMA] * 2
        ),
    )
    return sc_degree, sc_agg


def _tc1_body(degp_ref, xw_ref, w1_ref, dinv_ref, u1_ref):
    dinv = lax.rsqrt(degp_ref[0] + degp_ref[1] + 1.0)
    dinv_ref[...] = dinv
    u1_ref[...] = jnp.dot(xw_ref[...], w1_ref[...],
                          preferred_element_type=jnp.float32) * dinv


def _tc_mid_body(sp_ref, u_ref, dinv_ref, b_ref, w_ref, unext_ref):
    dinv = dinv_ref[...]
    h = jnp.maximum(dinv * (sp_ref[0] + sp_ref[1] + u_ref[...]) + b_ref[...],
                    0.0)
    unext_ref[...] = jnp.dot(h, w_ref[...],
                             preferred_element_type=jnp.float32) * dinv


def _tc_final_body(sp_ref, u_ref, dinv_ref, b3_ref, batch4_ref,
                   gw1_ref, gb1_ref, gw2_ref, gb2_ref, tw_ref, tb_ref,
                   nw1_ref, nb1_ref, nw2_ref, nb2_ref, nw2c_ref, nb2c_ref,
                   bw1_ref, bb1_ref, bw2_ref, bb2_ref,
                   t_ref, n4_ref, bout_ref):
    h = jnp.maximum(
        dinv_ref[...] * (sp_ref[0] + sp_ref[1] + u_ref[...]) + b3_ref[...],
        0.0)

    nmid = jnp.maximum(
        jnp.dot(h, nw1_ref[...], preferred_element_type=jnp.float32)
        + nb1_ref[...], 0.0)
    nbig = jnp.maximum(
        jnp.dot(nmid, nw2_ref[...], preferred_element_type=jnp.float32)
        + nb2_ref[...], 0.0)
    n4_ref[...] = jnp.maximum(
        jnp.dot(nmid, nw2c_ref[...], preferred_element_type=jnp.float32)
        + nb2c_ref[...], 0.0)

    iota_b = lax.broadcasted_iota(jnp.int32, (_B, _ROWS), 0)
    sums = None
    bsums = None
    counts = None
    nh = nbig * h
    for q in range(_PK):
        oh = (iota_b == batch4_ref[q:q + 1, :]).astype(jnp.float32)
        sq = jnp.dot(oh, h, preferred_element_type=jnp.float32)
        bq = jnp.dot(oh, nh, preferred_element_type=jnp.float32)
        cq = jnp.sum(oh, axis=1, keepdims=True)
        sq = sq[:, q * _F:(q + 1) * _F]
        bq = bq[:, q * _F:(q + 1) * _F]
        sums = sq if sums is None else sums + sq
        bsums = bq if bsums is None else bsums + bq
        counts = cq if counts is None else counts + cq

    g = sums / jnp.maximum(counts, 1.0)
    g = jnp.dot(g, gw1_ref[...], preferred_element_type=jnp.float32) + gb1_ref[...]
    g = jnp.dot(g, gw2_ref[...], preferred_element_type=jnp.float32) + gb2_ref[...]
    t_ref[...] = jnp.maximum(
        jnp.dot(g, tw_ref[...], preferred_element_type=jnp.float32)
        + tb_ref[...], 0.0)

    bb = jnp.maximum(
        jnp.dot(bsums, bw1_ref[...], preferred_element_type=jnp.float32)
        + bb1_ref[...], 0.0)
    bout_ref[...] = jnp.maximum(
        jnp.dot(bb, bw2_ref[...], preferred_element_type=jnp.float32)
        + bb2_ref[...], 0.0)


_tc1 = pl.pallas_call(
    _tc1_body,
    out_shape=[jax.ShapeDtypeStruct((_ROWS, 128), jnp.float32),
               jax.ShapeDtypeStruct((_ROWS, 128), jnp.float32)],
)

_tc_mid = pl.pallas_call(
    _tc_mid_body,
    out_shape=jax.ShapeDtypeStruct((_ROWS, 128), jnp.float32),
)

_tc_final = pl.pallas_call(
    _tc_final_body,
    out_shape=[jax.ShapeDtypeStruct((_B, 2), jnp.float32),
               jax.ShapeDtypeStruct((_ROWS, _PK), jnp.float32),
               jax.ShapeDtypeStruct((_B, 3), jnp.float32)],
)


def _pad_cols(a, width):
    return jnp.pad(a, ((0, 0), (0, width - a.shape[1])))


def _pad_rows(a, height):
    return jnp.pad(a, ((0, height - a.shape[0]), (0, 0)))


def _blockdiag(w):
    return jnp.kron(jnp.eye(_PK, dtype=w.dtype), w)


def kernel(x, edge_index, batch, W1, b1, W2, b2, W3, b3, gW1, gb1, gW2, gb2,
           tW, tb, nW1, nb1, nW2, nb2, bW1, bb1, bW2, bb2):
    f32 = jnp.float32
    edge3 = edge_index.reshape(2, _NCHUNKS, _CHUNK)
    xw = jnp.pad(x.reshape(_N // _PK, _PK * 128),
                 ((0, (_NP - _N) // _PK), (0, 0)))
    batch4 = jnp.pad(batch, (0, _NP - _N), constant_values=_B) \
        .reshape(_ROWS, _PK).T

    W1k = _blockdiag(W1)
    W2k = _blockdiag(W2)
    W3k = _blockdiag(_pad_cols(W3, _F))
    b1t = jnp.tile(b1, _PK).reshape(1, 128)
    b2t = jnp.tile(b2, _PK).reshape(1, 128)
    b3t = jnp.tile(_pad_cols(b3.reshape(1, -1), _F), (1, _PK))
    nW1k = _blockdiag(_pad_rows(nW1, _F))
    nb1t = jnp.tile(nb1, _PK).reshape(1, 64)
    nW2k = _blockdiag(nW2 @ jnp.ones((1, _F), f32))
    nb2t = jnp.full((1, 128), nb2[0], f32)
    nW2c = _blockdiag(nW2)
    nb2c = jnp.full((1, _PK), nb2[0], f32)
    gW1p = _pad_rows(gW1, _F)
    bW1p = _pad_rows(bW1, _F)

    _sc_degree, _sc_agg = _sc_kernels()

    deg_parts = _sc_degree(edge3)
    degf = deg_parts.reshape(2, _ROWS, 128)
    dinv, u1 = _tc1(degf, xw, W1k)
    s1 = _sc_agg(u1.reshape(_NP, _F), edge3).reshape(2, _ROWS, 128)
    u2 = _tc_mid(s1, u1, dinv, b1t, W2k)
    s2 = _sc_agg(u2.reshape(_NP, _F), edge3).reshape(2, _ROWS, 128)
    u3 = _tc_mid(s2, u2, dinv, b2t, W3k)
    s3 = _sc_agg(u3.reshape(_NP, _F), edge3).reshape(2, _ROWS, 128)
    t, n4, b = _tc_final(
        s3, u3, dinv, b3t, batch4,
        gW1p, gb1.reshape(1, -1), gW2, gb2.reshape(1, -1),
        tW, tb.reshape(1, -1),
        nW1k, nb1t, nW2k, nb2t, nW2c, nb2c,
        bW1p, bb1.reshape(1, -1), bW2, bb2.reshape(1, -1))
    n = n4.reshape(_NP, 1)[:_N]
    return (t, n, b)


# speedup vs baseline: 1.1557x; 1.1557x over previous
import functools

import jax
import jax.numpy as jnp
from jax import lax
from jax.experimental import pallas as pl
from jax.experimental.pallas import tpu as pltpu
from jax.experimental.pallas import tpu_sc as plsc

_N = 10000
_NP = 10240
_E = 320000
_B = 64
_F = 32
_PK = 4
_ROWS = _NP // _PK
_CHUNK = 128
_NCHUNKS = _E // _CHUNK
_NC = 2
_NS = 16
_NW = _NC * _NS
_CPT = _NCHUNKS // _NW
_XTRA = _NCHUNKS - _CPT * _NW
_K = 13
_NGROUPS = _CPT // _K
_RPT = _NP // _NS


def _zero_rows(ref, nrows, width):
    z = jnp.zeros((16,), jnp.float32)

    def body(i, _):
        for j in range(width // 16):
            ref[i, pl.ds(j * 16, 16)] = z
        return 0

    lax.fori_loop(0, nrows, body, 0)


def _acc_zero(stage, acc, s):
    row0 = pl.multiple_of(s * _RPT, 8)
    pltpu.sync_copy(stage, acc.at[pl.ds(row0, _RPT)])


def _acc_writeback(stage, acc, out_hbm, c, s):
    row0 = pl.multiple_of(s * _RPT, 8)
    pltpu.sync_copy(acc.at[pl.ds(row0, _RPT)], stage)
    pltpu.sync_copy(stage, out_hbm.at[c, pl.ds(row0, _RPT)])


def _load_tile_indices(edge3_hbm, dim, idx2d, w):
    lo = w * _CPT
    pltpu.sync_copy(edge3_hbm.at[dim, pl.ds(lo, _CPT)],
                    idx2d.at[pl.ds(0, _CPT)])

    @pl.when(w < _XTRA)
    def _():
        pltpu.sync_copy(edge3_hbm.at[dim, pl.ds(_CPT * _NW + w, 1)],
                        idx2d.at[pl.ds(_CPT, 1)])


def _sc_degree_body(edge3_hbm, out_hbm, acc, dst2d, onesv, buf16, stage, ssem):
    c = lax.axis_index("c")
    s = lax.axis_index("s")
    w = c * _NS + s

    one = jnp.ones((16,), jnp.float32)

    def fill_ones(i, _):
        onesv[i] = one
        return 0

    lax.fori_loop(0, _CHUNK, fill_ones, 0)
    _zero_rows(buf16, _RPT, 16)
    row0 = pl.multiple_of(s * _RPT, 8)
    pltpu.sync_copy(buf16, acc.at[pl.ds(row0, _RPT)])
    _load_tile_indices(edge3_hbm, 1, dst2d, w)
    plsc.subcore_barrier()

    def group(g, _):
        descs = [pltpu.async_copy(onesv, acc.at[dst2d.at[g * _K + b]], ssem,
                                  add=True)
                 for b in range(_K)]
        for d in descs:
            d.wait()
        return 0

    lax.fori_loop(0, _NGROUPS, group, 0)

    @pl.when(w < _XTRA)
    def _():
        pltpu.sync_copy(onesv, acc.at[dst2d.at[_CPT]], add=True)

    plsc.subcore_barrier()
    pltpu.sync_copy(acc.at[pl.ds(row0, _RPT)], buf16)

    def expand(i, _):
        v = buf16[i]
        stage[i, pl.ds(0, 16)] = v
        stage[i, pl.ds(16, 16)] = v
        return 0

    lax.fori_loop(0, _RPT, expand, 0)
    pltpu.sync_copy(stage, out_hbm.at[c, pl.ds(row0, _RPT)])


def _sc_agg_body(u_hbm, edge3_hbm, out_hbm, acc, src2d, dst2d,
                 *rest):
    rows = rest[:_K]
    stage, gsem, ssem = rest[_K], rest[_K + 1], rest[_K + 2]
    c = lax.axis_index("c")
    s = lax.axis_index("s")
    w = c * _NS + s

    _load_tile_indices(edge3_hbm, 0, src2d, w)
    _load_tile_indices(edge3_hbm, 1, dst2d, w)
    for b in range(_K):
        pltpu.async_copy(u_hbm.at[src2d.at[b]], rows[b], gsem)
    _zero_rows(stage, _RPT, _F)
    _acc_zero(stage, acc, s)
    plsc.subcore_barrier()

    def _gwait(buf):
        pltpu.make_async_copy(u_hbm.at[src2d.at[0]], buf, gsem).wait()

    def _swait(buf):
        pltpu.make_async_copy(buf, acc.at[dst2d.at[0]], ssem).wait()

    for b in range(_K):
        _gwait(rows[b])
        pltpu.async_copy(rows[b], acc.at[dst2d.at[b]], ssem, add=True)

    def group(g, _):
        for b in range(_K):
            _swait(rows[b])
            pltpu.async_copy(u_hbm.at[src2d.at[g * _K + b]], rows[b], gsem)
        for b in range(_K):
            _gwait(rows[b])
            pltpu.async_copy(rows[b], acc.at[dst2d.at[g * _K + b]],
                             ssem, add=True)
        return 0

    lax.fori_loop(1, _NGROUPS, group, 0)
    for b in range(_K):
        _swait(rows[b])

    @pl.when(w < _XTRA)
    def _():
        pltpu.async_copy(u_hbm.at[src2d.at[_CPT]], rows[0], gsem).wait()
        pltpu.sync_copy(rows[0], acc.at[dst2d.at[_CPT]], add=True)

    plsc.subcore_barrier()
    _acc_writeback(stage, acc, out_hbm, c, s)


@functools.lru_cache(maxsize=None)
def _sc_kernels():
    mesh = plsc.VectorSubcoreMesh(core_axis_name="c", subcore_axis_name="s",
                                  num_cores=_NC, num_subcores=_NS)
    sc_degree = pl.kernel(
        _sc_degree_body,
        out_type=jax.ShapeDtypeStruct((_NC, _NP, _F), jnp.float32),
        mesh=mesh,
        compiler_params=pltpu.CompilerParams(use_tc_tiling_on_sc=False),
        scratch_types=[
            pltpu.VMEM_SHARED((_NP, 16), jnp.float32),
            pltpu.VMEM((_CPT + 1, _CHUNK), jnp.int32),
            pltpu.VMEM((_CHUNK, 16), jnp.float32),
            pltpu.VMEM((_RPT, 16), jnp.float32),
            pltpu.VMEM((_RPT, _F), jnp.float32),
            pltpu.SemaphoreType.DMA,
        ],
    )
    sc_agg = pl.kernel(
        _sc_agg_body,
        out_type=jax.ShapeDtypeStruct((_NC, _NP, _F), jnp.float32),
        mesh=mesh,
        compiler_params=pltpu.CompilerParams(use_tc_tiling_on_sc=False),
        scratch_types=(
            [pltpu.VMEM_SHARED((_NP, _F), jnp.float32)]
            + [pltpu.VMEM((_CPT + 1, _CHUNK), jnp.int32)] * 2
            + [pltpu.VMEM((_CHUNK, _F), jnp.float32)] * _K
            + [pltpu.VMEM((_RPT, _F), jnp.float32)]
            + [pltpu.SemaphoreType.DMA] * 2
        ),
    )
    return sc_degree, sc_agg


def _tc_mm_body(xw_ref, w1_ref, xw1_ref):
    xw1_ref[...] = jnp.dot(xw_ref[...], w1_ref[...],
                           preferred_element_type=jnp.float32)


def _tc1_body(degp_ref, xw1_ref, dinv_ref, u1_ref):
    dinv = lax.rsqrt(degp_ref[0] + degp_ref[1] + 1.0)
    dinv_ref[...] = dinv
    u1_ref[...] = xw1_ref[...] * dinv


def _tc_mid_body(sp_ref, u_ref, dinv_ref, b_ref, w_ref, unext_ref):
    dinv = dinv_ref[...]
    h = jnp.maximum(dinv * (sp_ref[0] + sp_ref[1] + u_ref[...]) + b_ref[...],
                    0.0)
    unext_ref[...] = jnp.dot(h, w_ref[...],
                             preferred_element_type=jnp.float32) * dinv


def _tc_final_body(sp_ref, u_ref, dinv_ref, b3_ref, batch4_ref,
                   gw1_ref, gb1_ref, gw2_ref, gb2_ref, tw_ref, tb_ref,
                   nw1_ref, nb1_ref, nw2_ref, nb2_ref, nw2c_ref, nb2c_ref,
                   bw1_ref, bb1_ref, bw2_ref, bb2_ref,
                   t_ref, n4_ref, bout_ref):
    h = jnp.maximum(
        dinv_ref[...] * (sp_ref[0] + sp_ref[1] + u_ref[...]) + b3_ref[...],
        0.0)

    nmid = jnp.maximum(
        jnp.dot(h, nw1_ref[...], preferred_element_type=jnp.float32)
        + nb1_ref[...], 0.0)
    nbig = jnp.maximum(
        jnp.dot(nmid, nw2_ref[...], preferred_element_type=jnp.float32)
        + nb2_ref[...], 0.0)
    n4_ref[...] = jnp.maximum(
        jnp.dot(nmid, nw2c_ref[...], preferred_element_type=jnp.float32)
        + nb2c_ref[...], 0.0)

    iota_b = lax.broadcasted_iota(jnp.int32, (_B, _ROWS), 0)
    sums = None
    bsums = None
    counts = None
    nh = nbig * h
    for q in range(_PK):
        oh = (iota_b == batch4_ref[q:q + 1, :]).astype(jnp.float32)
        sq = jnp.dot(oh, h, preferred_element_type=jnp.float32)
        bq = jnp.dot(oh, nh, preferred_element_type=jnp.float32)
        cq = jnp.sum(oh, axis=1, keepdims=True)
        sq = sq[:, q * _F:(q + 1) * _F]
        bq = bq[:, q * _F:(q + 1) * _F]
        sums = sq if sums is None else sums + sq
        bsums = bq if bsums is None else bsums + bq
        counts = cq if counts is None else counts + cq

    g = sums / jnp.maximum(counts, 1.0)
    g = jnp.dot(g, gw1_ref[...], preferred_element_type=jnp.float32) + gb1_ref[...]
    g = jnp.dot(g, gw2_ref[...], preferred_element_type=jnp.float32) + gb2_ref[...]
    t_ref[...] = jnp.maximum(
        jnp.dot(g, tw_ref[...], preferred_element_type=jnp.float32)
        + tb_ref[...], 0.0)

    bb = jnp.maximum(
        jnp.dot(bsums, bw1_ref[...], preferred_element_type=jnp.float32)
        + bb1_ref[...], 0.0)
    bout_ref[...] = jnp.maximum(
        jnp.dot(bb, bw2_ref[...], preferred_element_type=jnp.float32)
        + bb2_ref[...], 0.0)


_tc_mm = pl.pallas_call(
    _tc_mm_body,
    out_shape=jax.ShapeDtypeStruct((_ROWS, 128), jnp.float32),
)

_tc1 = pl.pallas_call(
    _tc1_body,
    out_shape=[jax.ShapeDtypeStruct((_ROWS, 128), jnp.float32),
               jax.ShapeDtypeStruct((_ROWS, 128), jnp.float32)],
)

_tc_mid = pl.pallas_call(
    _tc_mid_body,
    out_shape=jax.ShapeDtypeStruct((_ROWS, 128), jnp.float32),
)

_tc_final = pl.pallas_call(
    _tc_final_body,
    out_shape=[jax.ShapeDtypeStruct((_B, 2), jnp.float32),
               jax.ShapeDtypeStruct((_ROWS, _PK), jnp.float32),
               jax.ShapeDtypeStruct((_B, 3), jnp.float32)],
)


def _pad_cols(a, width):
    return jnp.pad(a, ((0, 0), (0, width - a.shape[1])))


def _pad_rows(a, height):
    return jnp.pad(a, ((0, height - a.shape[0]), (0, 0)))


def _blockdiag(w):
    return jnp.kron(jnp.eye(_PK, dtype=w.dtype), w)


def kernel(x, edge_index, batch, W1, b1, W2, b2, W3, b3, gW1, gb1, gW2, gb2,
           tW, tb, nW1, nb1, nW2, nb2, bW1, bb1, bW2, bb2):
    f32 = jnp.float32
    edge3 = edge_index.reshape(2, _NCHUNKS, _CHUNK)
    xw = jnp.pad(x.reshape(_N // _PK, _PK * 128),
                 ((0, (_NP - _N) // _PK), (0, 0)))
    batch4 = jnp.pad(batch, (0, _NP - _N), constant_values=_B) \
        .reshape(_ROWS, _PK).T

    W1k = _blockdiag(W1)
    W2k = _blockdiag(W2)
    W3k = _blockdiag(_pad_cols(W3, _F))
    b1t = jnp.tile(b1, _PK).reshape(1, 128)
    b2t = jnp.tile(b2, _PK).reshape(1, 128)
    b3t = jnp.tile(_pad_cols(b3.reshape(1, -1), _F), (1, _PK))
    nW1k = _blockdiag(_pad_rows(nW1, _F))
    nb1t = jnp.tile(nb1, _PK).reshape(1, 64)
    nW2k = _blockdiag(nW2 @ jnp.ones((1, _F), f32))
    nb2t = jnp.full((1, 128), nb2[0], f32)
    nW2c = _blockdiag(nW2)
    nb2c = jnp.full((1, _PK), nb2[0], f32)
    gW1p = _pad_rows(gW1, _F)
    bW1p = _pad_rows(bW1, _F)

    _sc_degree, _sc_agg = _sc_kernels()

    xw1 = _tc_mm(xw, W1k)
    deg_parts = _sc_degree(edge3)
    degf = deg_parts.reshape(2, _ROWS, 128)
    dinv, u1 = _tc1(degf, xw1)
    s1 = _sc_agg(u1.reshape(_NP, _F), edge3).reshape(2, _ROWS, 128)
    u2 = _tc_mid(s1, u1, dinv, b1t, W2k)
    s2 = _sc_agg(u2.reshape(_NP, _F), edge3).reshape(2, _ROWS, 128)
    u3 = _tc_mid(s2, u2, dinv, b2t, W3k)
    s3 = _sc_agg(u3.reshape(_NP, _F), edge3).reshape(2, _ROWS, 128)
    t, n4, b = _tc_final(
        s3, u3, dinv, b3t, batch4,
        gW1p, gb1.reshape(1, -1), gW2, gb2.reshape(1, -1),
        tW, tb.reshape(1, -1),
        nW1k, nb1t, nW2k, nb2t, nW2c, nb2c,
        bW1p, bb1.reshape(1, -1), bW2, bb2.reshape(1, -1))
    n = n4.reshape(_NP, 1)[:_N]
    return (t, n, b)
